# baseline (speedup 1.0000x reference)
import functools

import jax
import jax.numpy as jnp
from jax import lax
from jax.experimental import pallas as pl
from jax.experimental.pallas import tpu as pltpu
from jax.experimental.pallas import tpu_sc as plsc

VOCAB = 1000000
EMBED_DIM = 64
CONTEXT_LEN = 200
BATCH = 4096
N_TOK = BATCH * CONTEXT_LEN

NUM_CORES = 2
NUM_SUBCORES = 16
NUM_WORKERS = NUM_CORES * NUM_SUBCORES
TOK_PER_WORKER = N_TOK // NUM_WORKERS
K = 256
CHUNKS_PER_WORKER = TOK_PER_WORKER // K
GROUPS = K // 16
NVEC = EMBED_DIM // 16


def _positional_encoding():
    pos = jnp.arange(1, CONTEXT_LEN + 1, dtype=jnp.float32)[:, None]
    i = jnp.arange(1, EMBED_DIM + 1, dtype=jnp.float32)[None, :]
    return 1.0 - pos / CONTEXT_LEN - (i / EMBED_DIM) * (1.0 - 2.0 * pos / CONTEXT_LEN)


def _sc_embed(table, cflat, lflat, rflat, pe):
    mesh = plsc.VectorSubcoreMesh(core_axis_name="c", subcore_axis_name="s")

    @functools.partial(
        pl.kernel,
        out_type=jax.ShapeDtypeStruct((N_TOK, EMBED_DIM), jnp.float32),
        mesh=mesh,
        compiler_params=pltpu.CompilerParams(use_tc_tiling_on_sc=False),
        scratch_types=[
            pltpu.VMEM((CONTEXT_LEN, EMBED_DIM), jnp.float32),
            pltpu.VMEM((K,), jnp.int32),
            pltpu.VMEM((K,), jnp.int32),
            pltpu.VMEM((K,), jnp.int32),
            pltpu.VMEM((K, EMBED_DIM), jnp.float32),
            pltpu.VMEM((K, EMBED_DIM), jnp.float32),
            pltpu.VMEM((K, EMBED_DIM), jnp.float32),
            pltpu.SemaphoreType.DMA,
        ],
    )
    def k(table_hbm, c_hbm, l_hbm, r_hbm, pe_hbm, out_hbm,
          pe_v, ci_v, li_v, ri_v, cbuf, lbuf, rbuf, sem):
        wid = lax.axis_index("s") * NUM_CORES + lax.axis_index("c")
        base = wid * TOK_PER_WORKER
        pltpu.sync_copy(pe_hbm, pe_v)

        def chunk(ch, carry):
            off = base + ch * K
            pltpu.sync_copy(c_hbm.at[pl.ds(off, K)], ci_v)
            pltpu.sync_copy(l_hbm.at[pl.ds(off, K)], li_v)
            pltpu.sync_copy(r_hbm.at[pl.ds(off, K)], ri_v)
            gc = pltpu.async_copy(table_hbm.at[ci_v], cbuf, sem)
            gl = pltpu.async_copy(table_hbm.at[li_v], lbuf, sem)
            gr = pltpu.async_copy(table_hbm.at[ri_v], rbuf, sem)
            gc.wait()
            gl.wait()
            gr.wait()
            pe_off = lax.rem(ch * K, CONTEXT_LEN)

            def group(g, carry2):
                mc16 = jnp.where(ci_v[pl.ds(g * 16, 16)] != 0, 1.0, 0.0)
                ml16 = jnp.where(li_v[pl.ds(g * 16, 16)] != 0, 1.0, 0.0)
                mr16 = jnp.where(ri_v[pl.ds(g * 16, 16)] != 0, 1.0, 0.0)
                for t in range(16):
                    i = g * 16 + t
                    p = lax.rem(pe_off + i, CONTEXT_LEN)
                    mc = mc16[t]
                    ml = ml16[t]
                    mr = mr16[t]
                    for j in range(NVEC):
                        sl = pl.ds(j * 16, 16)
                        v = (pe_v[p, sl]
                             + mc * cbuf[i, sl]
                             + ml * lbuf[i, sl]
                             + mr * rbuf[i, sl])
                        cbuf[i, sl] = v
                return carry2

            lax.fori_loop(0, GROUPS, group, 0)
            pltpu.sync_copy(cbuf, out_hbm.at[pl.ds(off, K)])
            return carry

        lax.fori_loop(0, CHUNKS_PER_WORKER, chunk, 0)

    return k(table, cflat, lflat, rflat, pe)


@jax.jit
def kernel(table, contexts, left_spc_masks, right_spc_masks):
    pe = _positional_encoding()
    out = _sc_embed(
        table,
        contexts.reshape(N_TOK),
        left_spc_masks.reshape(N_TOK),
        right_spc_masks.reshape(N_TOK),
        pe,
    )
    return out.reshape(BATCH, CONTEXT_LEN, EMBED_DIM)


# speedup vs baseline: 1.0928x; 1.0928x over previous
import functools

import jax
import jax.numpy as jnp
from jax import lax
from jax.experimental import pallas as pl
from jax.experimental.pallas import tpu as pltpu
from jax.experimental.pallas import tpu_sc as plsc

VOCAB = 1000000
EMBED_DIM = 64
CONTEXT_LEN = 200
BATCH = 4096
N_TOK = BATCH * CONTEXT_LEN

NUM_CORES = 2
NUM_SUBCORES = 16
NUM_WORKERS = NUM_CORES * NUM_SUBCORES
TOK_PER_WORKER = N_TOK // NUM_WORKERS
K = 256
CPW = TOK_PER_WORKER // K
GROUPS = K // 16
NVEC = EMBED_DIM // 16


def _positional_encoding():
    pos = jnp.arange(1, CONTEXT_LEN + 1, dtype=jnp.float32)[:, None]
    i = jnp.arange(1, EMBED_DIM + 1, dtype=jnp.float32)[None, :]
    return 1.0 - pos / CONTEXT_LEN - (i / EMBED_DIM) * (1.0 - 2.0 * pos / CONTEXT_LEN)


def _sc_embed(table, cflat, lflat, rflat, pe):
    mesh = plsc.VectorSubcoreMesh(core_axis_name="c", subcore_axis_name="s")

    @functools.partial(
        pl.kernel,
        out_type=jax.ShapeDtypeStruct((N_TOK, EMBED_DIM), jnp.float32),
        mesh=mesh,
        compiler_params=pltpu.CompilerParams(use_tc_tiling_on_sc=False),
        scratch_types=[
            pltpu.VMEM((CONTEXT_LEN, EMBED_DIM), jnp.float32),
            pltpu.VMEM((2, K), jnp.int32),
            pltpu.VMEM((2, K), jnp.int32),
            pltpu.VMEM((2, K), jnp.int32),
            pltpu.VMEM((K, EMBED_DIM), jnp.float32),
            pltpu.VMEM((K, EMBED_DIM), jnp.float32),
            pltpu.VMEM((K, EMBED_DIM), jnp.float32),
            pltpu.VMEM((K, EMBED_DIM), jnp.float32),
            pltpu.VMEM((K, EMBED_DIM), jnp.float32),
            pltpu.VMEM((K, EMBED_DIM), jnp.float32),
            pltpu.SemaphoreType.DMA,
            pltpu.SemaphoreType.DMA,
            pltpu.SemaphoreType.DMA,
            pltpu.SemaphoreType.DMA,
            pltpu.SemaphoreType.DMA,
            pltpu.SemaphoreType.DMA,
        ],
    )
    def k(table_hbm, c_hbm, l_hbm, r_hbm, pe_hbm, out_hbm,
          pe_v, idx_c, idx_l, idx_r,
          ac, al, ar, bc, bl, br,
          semA, semB, semI0, semI1, osemA, osemB):
        wid = lax.axis_index("s") * NUM_CORES + lax.axis_index("c")
        base = wid * TOK_PER_WORKER
        pltpu.sync_copy(pe_hbm, pe_v)

        def load_idx_sync(c, slot):
            off = base + c * K
            pltpu.sync_copy(c_hbm.at[pl.ds(off, K)], idx_c.at[slot])
            pltpu.sync_copy(l_hbm.at[pl.ds(off, K)], idx_l.at[slot])
            pltpu.sync_copy(r_hbm.at[pl.ds(off, K)], idx_r.at[slot])

        def issue_gathers(c, slot, xc, xl, xr, sem):
            pltpu.async_copy(table_hbm.at[idx_c.at[slot]], xc, sem)
            pltpu.async_copy(table_hbm.at[idx_l.at[slot]], xl, sem)
            pltpu.async_copy(table_hbm.at[idx_r.at[slot]], xr, sem)

        def drain_gathers(slot, xc, xl, xr, sem):
            pltpu.make_async_copy(table_hbm.at[idx_c.at[slot]], xc, sem).wait()
            pltpu.make_async_copy(table_hbm.at[idx_l.at[slot]], xl, sem).wait()
            pltpu.make_async_copy(table_hbm.at[idx_r.at[slot]], xr, sem).wait()

        def compute(c, slot, xc, xl, xr):
            pe_off = lax.rem(c * K, CONTEXT_LEN)

            def group(g, carry):
                mc16 = jnp.where(idx_c[slot, pl.ds(g * 16, 16)] != 0, 1.0, 0.0)
                ml16 = jnp.where(idx_l[slot, pl.ds(g * 16, 16)] != 0, 1.0, 0.0)
                mr16 = jnp.where(idx_r[slot, pl.ds(g * 16, 16)] != 0, 1.0, 0.0)
                for t in range(16):
                    i = g * 16 + t
                    p = lax.rem(pe_off + i, CONTEXT_LEN)
                    mc = mc16[t]
                    ml = ml16[t]
                    mr = mr16[t]
                    for j in range(NVEC):
                        sl = pl.ds(j * 16, 16)
                        v = (pe_v[p, sl]
                             + mc * xc[i, sl]
                             + ml * xl[i, sl]
                             + mr * xr[i, sl])
                        xc[i, sl] = v
                return carry

            lax.fori_loop(0, GROUPS, group, 0)

        def half(c, slot, slot_o, xc, xl, xr, oc, ol, orr,
                 sem, sem_o, semI, semI_o, osem, osem_o):
            drain_gathers(slot, xc, xl, xr, sem)
            compute(c, slot, xc, xl, xr)
            @pl.when(c < CPW - 2)
            def _():
                off2 = base + (c + 2) * K
                pltpu.async_copy(c_hbm.at[pl.ds(off2, K)], idx_c.at[slot], semI)
                pltpu.async_copy(l_hbm.at[pl.ds(off2, K)], idx_l.at[slot], semI)
                pltpu.async_copy(r_hbm.at[pl.ds(off2, K)], idx_r.at[slot], semI)
            off = base + c * K
            pltpu.async_copy(xc, out_hbm.at[pl.ds(off, K)], osem)
            @pl.when(c >= 1)
            def _():
                offp = base + (c - 1) * K
                pltpu.make_async_copy(oc, out_hbm.at[pl.ds(offp, K)], osem_o).wait()
            @pl.when(c + 1 < CPW)
            def _():
                @pl.when(c >= 1)
                def _():
                    off1 = base + (c + 1) * K
                    pltpu.make_async_copy(c_hbm.at[pl.ds(off1, K)], idx_c.at[slot_o], semI_o).wait()
                    pltpu.make_async_copy(l_hbm.at[pl.ds(off1, K)], idx_l.at[slot_o], semI_o).wait()
                    pltpu.make_async_copy(r_hbm.at[pl.ds(off1, K)], idx_r.at[slot_o], semI_o).wait()
                issue_gathers(c + 1, slot_o, oc, ol, orr, sem_o)

        load_idx_sync(0, 0)
        load_idx_sync(1, 1)
        issue_gathers(0, 0, ac, al, ar, semA)

        def pair(kk, carry):
            c0 = 2 * kk
            half(c0, 0, 1, ac, al, ar, bc, bl, br,
                 semA, semB, semI0, semI1, osemA, osemB)
            half(c0 + 1, 1, 0, bc, bl, br, ac, al, ar,
                 semB, semA, semI1, semI0, osemB, osemA)
            return carry

        lax.fori_loop(0, CPW // 2, pair, 0)
        offl = base + (CPW - 1) * K
        pltpu.make_async_copy(bc, out_hbm.at[pl.ds(offl, K)], osemB).wait()

    return k(table, cflat, lflat, rflat, pe)


@jax.jit
def kernel(table, contexts, left_spc_masks, right_spc_masks):
    pe = _positional_encoding()
    out = _sc_embed(
        table,
        contexts.reshape(N_TOK),
        left_spc_masks.reshape(N_TOK),
        right_spc_masks.reshape(N_TOK),
        pe,
    )
    return out.reshape(BATCH, CONTEXT_LEN, EMBED_DIM)
